# baseline (speedup 1.0000x reference)
import functools

import jax
import jax.numpy as jnp
from jax import lax
from jax.experimental import pallas as pl
from jax.experimental.pallas import tpu as pltpu
from jax.experimental.pallas import tpu_sc as plsc

RAC_SIZE = 1000000
N_RASU = 2
GRID = 101
N_REFLN = 500000

NUM_CORES = 2
NUM_SUBCORES = 16
NW = NUM_CORES * NUM_SUBCORES
ROW = 128
ROWS_PER_W = 125
PER_W = ROWS_PER_W * ROW
PAD_N = NW * PER_W
GROUPS = PER_W // 16

_G1 = GRID
_G2 = GRID * GRID
_G3 = GRID * GRID * GRID


def _sc_body(rasu_hbm, hf_hbm, grid_hbm, out_hbm,
             rasu_v, hf_v, idx_v, obs_v, ones_v, sem_g, sem_s):
    c = lax.axis_index("c")
    s = lax.axis_index("s")
    wid = s * NUM_CORES + c
    base = wid * PER_W

    pltpu.sync_copy(rasu_hbm.at[pl.ds(base, PER_W)], rasu_v)
    pltpu.sync_copy(hf_hbm.at[pl.ds(base * 3, PER_W * 3)], hf_v)

    one16 = jnp.full((16,), 1.0, dtype=jnp.float32)
    for t in range(ROW // 16):
        ones_v[pl.ds(16 * t, 16)] = one16

    iota3 = lax.iota(jnp.int32, 16) * 3

    def grp(j, carry):
        p = j * 48 + iota3
        h0 = plsc.load_gather(hf_v, [p])
        h1 = plsc.load_gather(hf_v, [p + 1])
        h2 = plsc.load_gather(hf_v, [p + 2])
        r = rasu_v[pl.ds(j * 16, 16)]
        flat = r * _G3 + h0 * _G2 + h1 * _G1 + h2
        idx_v[pl.ds(j * 16, 16)] = flat
        return carry

    lax.fori_loop(0, GROUPS, grp, 0)

    def row(j, carry):
        pltpu.async_copy(grid_hbm.at[idx_v.at[pl.ds(j * ROW, ROW)]],
                         obs_v.at[j], sem_g).wait()
        pltpu.async_copy(ones_v, out_hbm.at[obs_v.at[j]], sem_s).wait()
        return carry

    lax.fori_loop(0, ROWS_PER_W, row, 0)


@functools.partial(jax.jit, donate_argnums=())
def _run(rasu_p, hf_p, grid_flat, observed):
    out_ref = jax.new_ref(observed)
    k = pl.kernel(
        _sc_body,
        out_type=(),
        mesh=plsc.VectorSubcoreMesh(
            core_axis_name="c", subcore_axis_name="s",
            num_cores=NUM_CORES, num_subcores=NUM_SUBCORES),
        compiler_params=pltpu.CompilerParams(needs_layout_passes=False),
        scratch_types=[
            pltpu.VMEM((PER_W,), jnp.int32),
            pltpu.VMEM((PER_W * 3,), jnp.int32),
            pltpu.VMEM((PER_W,), jnp.int32),
            pltpu.VMEM((ROWS_PER_W, ROW), jnp.int32),
            pltpu.VMEM((ROW,), jnp.float32),
            pltpu.SemaphoreType.DMA,
            pltpu.SemaphoreType.DMA,
        ],
    )
    k(rasu_p, hf_p, grid_flat, out_ref)
    return out_ref[...]


def kernel(observed, rasu_id, H, reflection_id_grid):
    pad = PAD_N - N_REFLN
    rasu_p = jnp.concatenate([rasu_id, jnp.broadcast_to(rasu_id[0], (pad,))])
    hf_p = jnp.concatenate([H.reshape(-1), jnp.tile(H[0], pad)])
    grid_flat = reflection_id_grid.reshape(-1)
    return _run(rasu_p, hf_p, grid_flat, observed)


# speedup vs baseline: 1.7850x; 1.7850x over previous
import jax
import jax.numpy as jnp
from jax import lax
from jax.experimental import pallas as pl
from jax.experimental.pallas import tpu as pltpu
from jax.experimental.pallas import tpu_sc as plsc

RAC_SIZE = 1000000
N_RASU = 2
GRID = 101
N_REFLN = 500000

NUM_CORES = 2
NUM_SUBCORES = 16
NW = NUM_CORES * NUM_SUBCORES
ROW = 128
W_ROWS = 123
W_ELEMS = W_ROWS * ROW
STRIDE = 15624
LAST_BASE = N_REFLN - W_ELEMS

_G1 = GRID
_G2 = GRID * GRID
_G3 = GRID * GRID * GRID


def _sc_body(rasu_hbm, hf_hbm, grid_hbm, out_hbm,
             rasu_v, hf_v, idx_v, obs_v, ones_v, sem_g, sem_s):
    c = lax.axis_index("c")
    s = lax.axis_index("s")
    wid = s * NUM_CORES + c
    base = jnp.minimum(wid * STRIDE, LAST_BASE)

    pltpu.sync_copy(rasu_hbm.at[pl.ds(base, W_ELEMS)], rasu_v)
    pltpu.sync_copy(hf_hbm.at[pl.ds(base * 3, W_ELEMS * 3)], hf_v)

    one16 = jnp.full((16,), 1.0, dtype=jnp.float32)
    for t in range(ROW // 16):
        ones_v[pl.ds(16 * t, 16)] = one16

    iota3 = lax.iota(jnp.int32, 16) * 3

    def compute_row(j, carry):
        for g in range(ROW // 16):
            p = j * (3 * ROW) + (48 * g) + iota3
            h0 = plsc.load_gather(hf_v, [p])
            h1 = plsc.load_gather(hf_v, [p + 1])
            h2 = plsc.load_gather(hf_v, [p + 2])
            r = rasu_v[pl.ds(j * ROW + 16 * g, 16)]
            flat = r * _G3 + h0 * _G2 + h1 * _G1 + h2
            idx_v[pl.ds(j * ROW + 16 * g, 16)] = flat
        pltpu.async_copy(grid_hbm.at[idx_v.at[pl.ds(j * ROW, ROW)]],
                         obs_v.at[j], sem_g)
        return carry

    lax.fori_loop(0, W_ROWS, compute_row, 0)

    def scatter_row(j, carry):
        pltpu.make_async_copy(grid_hbm.at[idx_v.at[pl.ds(j * ROW, ROW)]],
                              obs_v.at[j], sem_g).wait()
        pltpu.async_copy(ones_v, out_hbm.at[obs_v.at[j]], sem_s)
        return carry

    lax.fori_loop(0, W_ROWS, scatter_row, 0)

    def drain_row(j, carry):
        pltpu.make_async_copy(ones_v, out_hbm.at[obs_v.at[j]], sem_s).wait()
        return carry

    lax.fori_loop(0, W_ROWS, drain_row, 0)


def _run(rasu, hf, grid_flat, observed):
    out_ref = jax.new_ref(observed)
    k = pl.kernel(
        _sc_body,
        out_type=(),
        mesh=plsc.VectorSubcoreMesh(
            core_axis_name="c", subcore_axis_name="s",
            num_cores=NUM_CORES, num_subcores=NUM_SUBCORES),
        compiler_params=pltpu.CompilerParams(needs_layout_passes=False),
        scratch_types=[
            pltpu.VMEM((W_ELEMS,), jnp.int32),
            pltpu.VMEM((W_ELEMS * 3,), jnp.int32),
            pltpu.VMEM((W_ELEMS,), jnp.int32),
            pltpu.VMEM((W_ROWS, ROW), jnp.int32),
            pltpu.VMEM((ROW,), jnp.float32),
            pltpu.SemaphoreType.DMA,
            pltpu.SemaphoreType.DMA,
        ],
    )
    k(rasu, hf, grid_flat, out_ref)
    return out_ref[...]


def kernel(observed, rasu_id, H, reflection_id_grid):
    hf = H.reshape(-1)
    grid_flat = reflection_id_grid.reshape(-1)
    return _run(rasu_id, hf, grid_flat, observed)


# speedup vs baseline: 6.6205x; 3.7089x over previous
import jax
import jax.numpy as jnp
from jax import lax
from jax.experimental import pallas as pl
from jax.experimental.pallas import tpu as pltpu
from jax.experimental.pallas import tpu_sc as plsc

RAC_SIZE = 1000000
N_RASU = 2
GRID = 101
N_REFLN = 500000

NUM_CORES = 2
NUM_SUBCORES = 16
NW = NUM_CORES * NUM_SUBCORES
ROW = 128
W_ROWS = 123
W_ELEMS = W_ROWS * ROW
STRIDE = 15624
LAST_BASE = N_REFLN - W_ELEMS

_G1 = GRID
_G2 = GRID * GRID
_G3 = GRID * GRID * GRID


def _sc_body(fidx_hbm, grid_hbm, out_hbm,
             idx_v, obs_v, ones_v, sem_g, sem_s):
    c = lax.axis_index("c")
    s = lax.axis_index("s")
    wid = s * NUM_CORES + c
    base = jnp.minimum(wid * STRIDE, LAST_BASE)

    one16 = jnp.full((16,), 1.0, dtype=jnp.float32)
    for t in range(ROW // 16):
        ones_v[pl.ds(16 * t, 16)] = one16

    pltpu.sync_copy(fidx_hbm.at[pl.ds(base, W_ELEMS)], idx_v)

    def gather_row(j, carry):
        pltpu.async_copy(grid_hbm.at[idx_v.at[pl.ds(j * ROW, ROW)]],
                         obs_v.at[j], sem_g)
        return carry

    lax.fori_loop(0, W_ROWS, gather_row, 0)

    def scatter_row(j, carry):
        pltpu.make_async_copy(grid_hbm.at[idx_v.at[pl.ds(j * ROW, ROW)]],
                              obs_v.at[j], sem_g).wait()
        pltpu.async_copy(ones_v, out_hbm.at[obs_v.at[j]], sem_s)
        return carry

    lax.fori_loop(0, W_ROWS, scatter_row, 0)

    def drain_row(j, carry):
        pltpu.make_async_copy(ones_v, out_hbm.at[obs_v.at[j]], sem_s).wait()
        return carry

    lax.fori_loop(0, W_ROWS, drain_row, 0)


def _run(fidx, grid_flat, observed):
    out_ref = jax.new_ref(observed)
    k = pl.kernel(
        _sc_body,
        out_type=(),
        mesh=plsc.VectorSubcoreMesh(
            core_axis_name="c", subcore_axis_name="s",
            num_cores=NUM_CORES, num_subcores=NUM_SUBCORES),
        compiler_params=pltpu.CompilerParams(needs_layout_passes=False),
        scratch_types=[
            pltpu.VMEM((W_ELEMS,), jnp.int32),
            pltpu.VMEM((W_ROWS, ROW), jnp.int32),
            pltpu.VMEM((ROW,), jnp.float32),
            pltpu.SemaphoreType.DMA,
            pltpu.SemaphoreType.DMA,
        ],
    )
    k(fidx, grid_flat, out_ref)
    return out_ref[...]


def kernel(observed, rasu_id, H, reflection_id_grid):
    fidx = (rasu_id * _G3 + H[:, 0] * _G2 + H[:, 1] * _G1 + H[:, 2])
    grid_flat = reflection_id_grid.reshape(-1)
    return _run(fidx, grid_flat, observed)


# speedup vs baseline: 6.6277x; 1.0011x over previous
import jax
import jax.numpy as jnp
from jax import lax
from jax.experimental import pallas as pl
from jax.experimental.pallas import tpu as pltpu
from jax.experimental.pallas import tpu_sc as plsc

RAC_SIZE = 1000000
N_RASU = 2
GRID = 101
N_REFLN = 500000

NUM_CORES = 2
NUM_SUBCORES = 16
NW = NUM_CORES * NUM_SUBCORES
ROW = 128
W_ROWS = 123
W_ELEMS = W_ROWS * ROW
STRIDE = 15624
LAST_BASE = N_REFLN - W_ELEMS

_G1 = GRID
_G2 = GRID * GRID
_G3 = GRID * GRID * GRID


def _sc_body(fidx_hbm, grid_hbm, out_hbm,
             idx_v, obs_v, ones_v, sem_g, sem_s):
    c = lax.axis_index("c")
    s = lax.axis_index("s")
    wid = s * NUM_CORES + c
    base = jnp.minimum(wid * STRIDE, LAST_BASE)

    one16 = jnp.full((16,), 1.0, dtype=jnp.float32)

    def fill_ones(j, carry):
        ones_v[pl.ds(j * 16, 16)] = one16
        return carry

    lax.fori_loop(0, W_ELEMS // 16, fill_ones, 0)

    pltpu.sync_copy(fidx_hbm.at[pl.ds(base, W_ELEMS)], idx_v)
    pltpu.async_copy(grid_hbm.at[idx_v], obs_v, sem_g).wait()
    pltpu.async_copy(ones_v, out_hbm.at[obs_v], sem_s).wait()


def _run(fidx, grid_flat, observed):
    out_ref = jax.new_ref(observed)
    k = pl.kernel(
        _sc_body,
        out_type=(),
        mesh=plsc.VectorSubcoreMesh(
            core_axis_name="c", subcore_axis_name="s",
            num_cores=NUM_CORES, num_subcores=NUM_SUBCORES),
        compiler_params=pltpu.CompilerParams(needs_layout_passes=False),
        scratch_types=[
            pltpu.VMEM((W_ELEMS,), jnp.int32),
            pltpu.VMEM((W_ELEMS,), jnp.int32),
            pltpu.VMEM((W_ELEMS,), jnp.float32),
            pltpu.SemaphoreType.DMA,
            pltpu.SemaphoreType.DMA,
        ],
    )
    k(fidx, grid_flat, out_ref)
    return out_ref[...]


def kernel(observed, rasu_id, H, reflection_id_grid):
    fidx = (rasu_id * _G3 + H[:, 0] * _G2 + H[:, 1] * _G1 + H[:, 2])
    grid_flat = reflection_id_grid.reshape(-1)
    return _run(fidx, grid_flat, observed)


# speedup vs baseline: 32.8598x; 4.9579x over previous
import jax
import jax.numpy as jnp
from jax import lax
from jax.experimental import pallas as pl
from jax.experimental.pallas import tpu as pltpu
from jax.experimental.pallas import tpu_sc as plsc

RAC_SIZE = 1000000
N_RASU = 2
GRID = 101
N_REFLN = 500000

NUM_CORES = 2
NUM_SUBCORES = 16
NW = NUM_CORES * NUM_SUBCORES

_G1 = GRID
_G2 = GRID * GRID
_G3 = GRID * GRID * GRID

W_ELEMS = 15744
STRIDE = 15624
LAST_BASE = N_REFLN - W_ELEMS

IMG_PAD = 1000448
IMG_SLICE = IMG_PAD // NUM_SUBCORES
ZCHUNK = IMG_SLICE // 4

M_ELEMS = 31312
M_STRIDE = 31248
M_LAST = RAC_SIZE - M_ELEMS


def _scatter_body(fidx_hbm, grid_hbm, flags_hbm,
                  idx_v, obs_v, ones_v, zbuf, shared, sem_g, sem_s):
    c = lax.axis_index("c")
    s = lax.axis_index("s")
    wid = s * NUM_CORES + c
    base = jnp.minimum(wid * STRIDE, LAST_BASE)

    one16 = jnp.full((16,), 1, dtype=jnp.int32)
    zero16 = jnp.zeros((16,), dtype=jnp.int32)

    def fill_ones(j, carry):
        ones_v[pl.ds(j * 16, 16)] = one16
        return carry

    lax.fori_loop(0, W_ELEMS // 16, fill_ones, 0)

    def fill_zeros(j, carry):
        zbuf[pl.ds(j * 16, 16)] = zero16
        return carry

    lax.fori_loop(0, ZCHUNK // 16, fill_zeros, 0)

    pltpu.sync_copy(fidx_hbm.at[pl.ds(base, W_ELEMS)], idx_v)
    gather = pltpu.async_copy(grid_hbm.at[idx_v], obs_v, sem_g)

    for k in range(4):
        pltpu.sync_copy(zbuf,
                        shared.at[pl.ds(s * IMG_SLICE + k * ZCHUNK, ZCHUNK)])
    gather.wait()
    plsc.subcore_barrier()

    pltpu.async_copy(ones_v, shared.at[obs_v], sem_s).wait()
    plsc.subcore_barrier()

    for k in range(4):
        pltpu.sync_copy(shared.at[pl.ds(s * IMG_SLICE + k * ZCHUNK, ZCHUNK)],
                        zbuf)
        pltpu.sync_copy(zbuf,
                        flags_hbm.at[pl.ds(c * IMG_PAD + s * IMG_SLICE
                                           + k * ZCHUNK, ZCHUNK)])


def _merge_body(flags_hbm, obs_hbm, out_hbm, f0_v, f1_v, ob_v):
    c = lax.axis_index("c")
    s = lax.axis_index("s")
    wid = s * NUM_CORES + c
    base = jnp.minimum(wid * M_STRIDE, M_LAST)

    pltpu.sync_copy(flags_hbm.at[pl.ds(base, M_ELEMS)], f0_v)
    pltpu.sync_copy(flags_hbm.at[pl.ds(IMG_PAD + base, M_ELEMS)], f1_v)
    pltpu.sync_copy(obs_hbm.at[pl.ds(base, M_ELEMS)], ob_v)

    one16f = jnp.full((16,), 1.0, dtype=jnp.float32)

    def step(j, carry):
        f = f0_v[pl.ds(j * 16, 16)] | f1_v[pl.ds(j * 16, 16)]
        o = ob_v[pl.ds(j * 16, 16)]
        ob_v[pl.ds(j * 16, 16)] = jnp.where(f != 0, one16f, o)
        return carry

    lax.fori_loop(0, M_ELEMS // 16, step, 0)

    pltpu.sync_copy(ob_v, out_hbm.at[pl.ds(base, M_ELEMS)])


def _mesh():
    return plsc.VectorSubcoreMesh(
        core_axis_name="c", subcore_axis_name="s",
        num_cores=NUM_CORES, num_subcores=NUM_SUBCORES)


def _run(fidx, grid_flat, observed):
    scatter_k = pl.kernel(
        _scatter_body,
        out_type=jax.ShapeDtypeStruct((2 * IMG_PAD,), jnp.int32),
        mesh=_mesh(),
        compiler_params=pltpu.CompilerParams(needs_layout_passes=False),
        scratch_types=[
            pltpu.VMEM((W_ELEMS,), jnp.int32),
            pltpu.VMEM((W_ELEMS,), jnp.int32),
            pltpu.VMEM((W_ELEMS,), jnp.int32),
            pltpu.VMEM((ZCHUNK,), jnp.int32),
            pltpu.VMEM_SHARED((IMG_PAD,), jnp.int32),
            pltpu.SemaphoreType.DMA,
            pltpu.SemaphoreType.DMA,
        ],
    )
    flags = scatter_k(fidx, grid_flat)

    merge_k = pl.kernel(
        _merge_body,
        out_type=jax.ShapeDtypeStruct((RAC_SIZE,), jnp.float32),
        mesh=_mesh(),
        compiler_params=pltpu.CompilerParams(needs_layout_passes=False),
        scratch_types=[
            pltpu.VMEM((M_ELEMS,), jnp.int32),
            pltpu.VMEM((M_ELEMS,), jnp.int32),
            pltpu.VMEM((M_ELEMS,), jnp.float32),
        ],
    )
    return merge_k(flags, observed)


def kernel(observed, rasu_id, H, reflection_id_grid):
    fidx = (rasu_id * _G3 + H[:, 0] * _G2 + H[:, 1] * _G1 + H[:, 2])
    grid_flat = reflection_id_grid.reshape(-1)
    return _run(fidx, grid_flat, observed)
